# baseline (speedup 1.0000x reference)
import functools

import jax
import jax.numpy as jnp
from jax import lax
from jax.experimental import pallas as pl
from jax.experimental.pallas import tpu as pltpu
from jax.experimental.pallas import tpu_sc as plsc

N_ENT = 10000
N_USERS = 20000
CH = 128
N_EDGES = 320000
NNZ = 500000
N_REL = 10
N_FACT = 4
N_HOPS = 2

NC = 2
NS = 16
L = 16
NW = NC * NS

KG_CPS = 160
E_PAD = NS * KG_CPS * 128
KGW = 72
ACC_E = 10240
UI_CPS = 248
NNZ_PAD = NS * UI_CPS * 128
ACC_U = 20480
WREP = 128
NB = 2

def _sc_mesh():
    return plsc.VectorSubcoreMesh(core_axis_name="c", subcore_axis_name="s",
                                  num_cores=NC, num_subcores=NS)


def _kg_body(ent2_h, pack_h, w2_h, out_h,
             pack, rows, stage, hbuf, ebuf, wtab, acc, *sems):
    c = lax.axis_index("c")
    s = lax.axis_index("s")
    pltpu.sync_copy(w2_h.at[c], wtab.at[pl.ds(0, N_REL - 1)])
    si, sg, ss = sems[0:NB], sems[NB:2 * NB], sems[2 * NB:3 * NB]
    z16 = jnp.zeros((L,), jnp.float32)
    iota16 = lax.iota(jnp.int32, L)

    @plsc.parallel_loop(0, 128, unroll=4)
    def _z1(r):
        for u in range(4):
            stage[0, r, pl.ds(u * L, L)] = z16
        stage[0, r, pl.ds(56, L)] = z16

    r0 = s * (ACC_E // NS)

    def z3(k, _):
        pltpu.sync_copy(stage.at[0], acc.at[pl.ds(r0 + k * 128, 128)])
        return 0
    lax.fori_loop(0, ACC_E // NS // 128, z3, 0)

    onehot = jnp.where(iota16 == 8, 1.0, 0.0).astype(jnp.float32)

    @plsc.parallel_loop(0, 128, unroll=8)
    def _z4(r):
        for b in range(NB):
            stage[b, r, pl.ds(56, L)] = onehot

    base = s * KG_CPS

    def issue_i(ci, b):
        pltpu.async_copy(pack_h.at[ci], pack.at[b], si[b])

    def wait_i(b):
        pltpu.make_async_copy(pack_h.at[0], pack.at[b], si[b]).wait()

    def fix_and_g(b, h):
        for g in range(8):
            sl = pl.ds(g * L, L)
            pack[b, 0, sl] = pack[b, 0, sl] + c
            hbuf[h, sl] = pack[b, 1, sl]
            ebuf[b, sl] = pack[b, 2, sl]
        pltpu.async_copy(ent2_h.at[pack.at[b].at[0]], rows.at[b], sg[b])

    def wait_g(b):
        pltpu.make_async_copy(ent2_h.at[pack.at[b].at[0]], rows.at[b],
                              sg[b]).wait()

    def compute_s(b, h):
        bv = jnp.full((L,), b, jnp.int32)
        for g in range(8):
            et16 = ebuf[b, pl.ds(g * L, L)]
            lid16 = iota16 + g * L

            @plsc.parallel_loop(0, CH // 2, unroll=4)
            def _m(ch):
                chv = jnp.full((L,), ch, jnp.int32)
                wv = plsc.load_gather(wtab, [et16, chv])
                gv = plsc.load_gather(rows, [bv, lid16, chv])
                plsc.store_scatter(stage, [bv, lid16, chv], wv * gv)
        pltpu.async_copy(stage.at[b], acc.at[hbuf.at[h]], ss[b], add=True)

    def wait_s(b, h):
        pltpu.make_async_copy(stage.at[b], acc.at[hbuf.at[h]], ss[b]).wait()

    issue_i(base, 0)
    plsc.subcore_barrier()
    wait_i(0)
    fix_and_g(0, 0)
    issue_i(base + 1, 1)

    def outer(jj, _):
        for b4 in range(4):
            j = jj * 4 + b4
            b = b4 % NB
            b1 = (b + 1) % NB
            h1 = (b4 + 1) % 4

            @pl.when(j + 1 < KG_CPS)
            def _():
                wait_i(b1)
                fix_and_g(b1, h1)
            wait_g(b)

            @pl.when(j + NB < KG_CPS)
            def _():
                issue_i(base + j + NB, b)

            @pl.when(j >= NB)
            def _():
                wait_s(b, b4)
            compute_s(b, b4)
        return 0
    lax.fori_loop(0, KG_CPS // 4, outer, 0)
    wait_s(0, 0)
    wait_s(1, 1)
    plsc.subcore_barrier()
    nrow = ACC_E // NS
    pltpu.sync_copy(acc.at[pl.ds(r0, nrow)], out_h.at[c, pl.ds(r0, nrow)])


@functools.lru_cache(maxsize=None)
def _get_kg_call():
  return pl.kernel(
    _kg_body,
    out_type=jax.ShapeDtypeStruct((NC, ACC_E, KGW), jnp.float32),
    mesh=_sc_mesh(),
    compiler_params=pltpu.CompilerParams(use_tc_tiling_on_sc=False,
                                         needs_layout_passes=False),
    scratch_types=[
        pltpu.VMEM((NB, 3, 128), jnp.int32),
        pltpu.VMEM((NB, 128, CH // 2), jnp.float32),
        pltpu.VMEM((NB, 128, KGW), jnp.float32),
        pltpu.VMEM((4, 128), jnp.int32),
        pltpu.VMEM((NB, 128), jnp.int32),
        pltpu.VMEM((16, CH // 2), jnp.float32),
        pltpu.VMEM_SHARED((ACC_E, KGW), jnp.float32),
    ] + [pltpu.SemaphoreType.DMA] * (3 * NB),
  )


def _kg_call(*args):
    return _get_kg_call()(*args)


def _ui_body(ent2_h, pack_h, out_h,
             pack, grows, stage, hbuf, vbuf, acc, *sems):
    c = lax.axis_index("c")
    s = lax.axis_index("s")
    si, sg, ss = sems[0:NB], sems[NB:2 * NB], sems[2 * NB:3 * NB]
    z16 = jnp.zeros((L,), jnp.float32)

    @plsc.parallel_loop(0, 128, unroll=4)
    def _z1(r):
        for u in range(4):
            stage[0, r, pl.ds(u * L, L)] = z16

    r0 = s * (ACC_U // NS)

    def z3(k, _):
        pltpu.sync_copy(stage.at[0], acc.at[pl.ds(r0 + k * 128, 128)])
        return 0
    lax.fori_loop(0, ACC_U // NS // 128, z3, 0)

    base = s * UI_CPS

    def issue_i(ci, b):
        pltpu.async_copy(pack_h.at[ci], pack.at[b], si[b])

    def wait_i(b):
        pltpu.make_async_copy(pack_h.at[0], pack.at[b], si[b]).wait()

    def fix_and_g(b, h):
        for g in range(8):
            sl = pl.ds(g * L, L)
            pack[b, 0, sl] = pack[b, 0, sl] + c
            hbuf[h, sl] = pack[b, 1, sl]
            vbuf[b, sl] = pack[b, 2, sl]
        pltpu.async_copy(ent2_h.at[pack.at[b].at[0]], grows.at[b], sg[b])

    def wait_g(b):
        pltpu.make_async_copy(ent2_h.at[pack.at[b].at[0]], grows.at[b],
                              sg[b]).wait()

    def compute_s(b, h):
        @plsc.parallel_loop(0, 128, unroll=4)
        def _m(e):
            bv = jnp.full((L,), b, jnp.int32)
            ev = jnp.full((L,), e, jnp.int32)
            v16 = plsc.bitcast(plsc.load_gather(vbuf, [bv, ev]), jnp.float32)
            for u in range(4):
                sl = pl.ds(u * L, L)
                stage[b, e, sl] = grows[b, e, sl] * v16
        pltpu.async_copy(stage.at[b], acc.at[hbuf.at[h]], ss[b], add=True)

    def wait_s(b, h):
        pltpu.make_async_copy(stage.at[b], acc.at[hbuf.at[h]], ss[b]).wait()

    issue_i(base, 0)
    wait_i(0)
    fix_and_g(0, 0)
    issue_i(base + 1, 1)
    plsc.subcore_barrier()

    def outer(jj, _):
        for b4 in range(4):
            j = jj * 4 + b4
            b = b4 % NB
            b1 = (b + 1) % NB
            h1 = (b4 + 1) % 4

            @pl.when(j + 1 < UI_CPS)
            def _():
                wait_i(b1)
                fix_and_g(b1, h1)
            wait_g(b)

            @pl.when(j + NB < UI_CPS)
            def _():
                issue_i(base + j + NB, b)

            @pl.when(j >= NB)
            def _():
                wait_s(b, b4)
            compute_s(b, b4)
        return 0
    lax.fori_loop(0, UI_CPS // 4, outer, 0)
    wait_s(0, 0)
    wait_s(1, 1)
    plsc.subcore_barrier()
    nrow = ACC_U // NS
    pltpu.sync_copy(acc.at[pl.ds(r0, nrow)], out_h.at[c, pl.ds(r0, nrow)])


@functools.lru_cache(maxsize=None)
def _get_ui_call():
  return pl.kernel(
    _ui_body,
    out_type=jax.ShapeDtypeStruct((NC, ACC_U, CH // 2), jnp.float32),
    mesh=_sc_mesh(),
    compiler_params=pltpu.CompilerParams(use_tc_tiling_on_sc=False,
                                         needs_layout_passes=False),
    scratch_types=[
        pltpu.VMEM((NB, 3, 128), jnp.int32),
        pltpu.VMEM((NB, 128, CH // 2), jnp.float32),
        pltpu.VMEM((NB, 128, CH // 2), jnp.float32),
        pltpu.VMEM((4, 128), jnp.int32),
        pltpu.VMEM((NB, 128), jnp.int32),
        pltpu.VMEM_SHARED((ACC_U, CH // 2), jnp.float32),
    ] + [pltpu.SemaphoreType.DMA] * (3 * NB),
  )


def _ui_call(*args):
    return _get_ui_call()(*args)


def _prep_body(d_ref, w_ref, cor_ref, dw_ref):
    d = d_ref[...]
    nrm = jnp.sqrt(jnp.sum(d * d, axis=1, keepdims=True))
    dn = d / jnp.maximum(nrm, 1e-12)
    sim = jnp.dot(dn, dn.T, preferred_element_type=jnp.float32)
    r8 = lax.broadcasted_iota(jnp.int32, (8, 8), 0)
    c8 = lax.broadcasted_iota(jnp.int32, (8, 8), 1)
    cor = jnp.sum(jnp.where(c8 > r8, sim, 0.0))
    rr = lax.broadcasted_iota(jnp.int32, (8, CH), 0)
    cc = lax.broadcasted_iota(jnp.int32, (8, CH), 1)
    cor_ref[...] = jnp.where((rr == 0) & (cc == 0), cor, 0.0)
    logits = jnp.where(cc < (N_REL - 1), d, -jnp.inf)
    m = jnp.max(logits, axis=1, keepdims=True)
    e = jnp.exp(logits - m)
    sm = e / jnp.sum(e, axis=1, keepdims=True)
    dw_ref[...] = jnp.dot(sm[:, :16], w_ref[...],
                          preferred_element_type=jnp.float32)


def _prep_call(d_pad, w_pad):
    return pl.pallas_call(
        _prep_body,
        out_shape=[jax.ShapeDtypeStruct((8, CH), jnp.float32),
                   jax.ShapeDtypeStruct((8, CH), jnp.float32)],
    )(d_pad, w_pad)


_EBLK = 1000
_UBLK = 1000


def _ent_epi_body(s0_ref, s1_ref, res_ref, ent_ref, out_ref):
    h0 = s0_ref[...]
    h1 = s1_ref[...]
    sums = jnp.concatenate([h0[:, :CH // 2], h1[:, :CH // 2]], axis=1)
    cnt = h0[:, CH // 2:CH // 2 + 1]
    agg = sums / jnp.maximum(cnt, 1.0)
    nrm = jnp.sqrt(jnp.sum(agg * agg, axis=1, keepdims=True))
    e = agg / jnp.maximum(nrm, 1e-12)
    ent_ref[...] = e
    out_ref[...] = res_ref[...] + e


def _ent_epi(s0, s1, res):
    g = N_ENT // _EBLK
    bs = lambda w: pl.BlockSpec((_EBLK, w), lambda i: (i, 0))
    return pl.pallas_call(
        _ent_epi_body,
        grid=(g,),
        in_specs=[bs(KGW), bs(KGW), bs(CH)],
        out_specs=[bs(CH), bs(CH)],
        out_shape=[jax.ShapeDtypeStruct((N_ENT, CH), jnp.float32),
                   jax.ShapeDtypeStruct((N_ENT, CH), jnp.float32)],
    )(s0, s1, res)


def _usr_epi_body(ua0_ref, ua1_ref, usr_ref, lat_ref, dw_ref, res_ref,
                  unew_ref, out_ref):
    ua = jnp.concatenate([ua0_ref[...], ua1_ref[...]], axis=1)
    logits = jnp.dot(usr_ref[...], lat_ref[...].T,
                     preferred_element_type=jnp.float32)
    c8 = lax.broadcasted_iota(jnp.int32, (_UBLK, 8), 1)
    lg = jnp.where(c8 < N_FACT, logits, -jnp.inf)
    m = jnp.max(lg, axis=1, keepdims=True)
    e = jnp.exp(lg - m)
    score = e / jnp.sum(e, axis=1, keepdims=True)
    factor = jnp.dot(score, dw_ref[...], preferred_element_type=jnp.float32)
    agg = ua * (1.0 + factor)
    nrm = jnp.sqrt(jnp.sum(agg * agg, axis=1, keepdims=True))
    u = agg / jnp.maximum(nrm, 1e-12)
    unew_ref[...] = u
    out_ref[...] = res_ref[...] + u


def _usr_epi(ua0, ua1, usr, lat_pad, dw, res):
    g = N_USERS // _UBLK
    bs = lambda w: pl.BlockSpec((_UBLK, w), lambda i: (i, 0))
    fs = pl.BlockSpec((8, CH), lambda i: (0, 0))
    return pl.pallas_call(
        _usr_epi_body,
        grid=(g,),
        in_specs=[bs(CH // 2), bs(CH // 2), bs(CH), fs, fs, bs(CH)],
        out_specs=[bs(CH), bs(CH)],
        out_shape=[jax.ShapeDtypeStruct((N_USERS, CH), jnp.float32),
                   jax.ShapeDtypeStruct((N_USERS, CH), jnp.float32)],
    )(ua0, ua1, usr, lat_pad, dw, res)


def kernel(user_emb, entity_emb, latent_emb, edge_index, edge_type,
           inter_edge, inter_edge_w, mat_rows, mat_cols, mat_vals,
           weight, disen_weight_att):
    f32 = jnp.float32
    i32 = jnp.int32

    epad = E_PAD - N_EDGES
    head_p = jnp.concatenate(
        [edge_index[0], N_ENT + (jnp.arange(epad, dtype=i32) % L)])
    tail_p = jnp.concatenate([edge_index[1], jnp.zeros((epad,), i32)])
    ety_p = jnp.concatenate([edge_type, jnp.ones((epad,), i32)])
    kg_pack = (jnp.stack([2 * tail_p, head_p, ety_p - 1], axis=0)
               .reshape(3, E_PAD // 128, 128).transpose(1, 0, 2))
    w2 = weight.reshape(N_REL - 1, 2, CH // 2).transpose(1, 0, 2)

    npad = NNZ_PAD - NNZ
    cols2_p = 2 * jnp.concatenate([mat_cols, jnp.zeros((npad,), i32)])
    urow_p = jnp.concatenate(
        [mat_rows, N_USERS + (jnp.arange(npad, dtype=i32) % L)])
    vals_p = jnp.concatenate([mat_vals, jnp.zeros((npad,), f32)])
    ui_pack = (jnp.stack([cols2_p, urow_p,
                          lax.bitcast_convert_type(vals_p, i32)], axis=0)
               .reshape(3, NNZ_PAD // 128, 128).transpose(1, 0, 2))

    d_pad = jnp.zeros((8, CH), f32).at[:N_FACT, :N_REL - 1].set(disen_weight_att)
    w_pad = jnp.zeros((16, CH), f32).at[:N_REL - 1].set(weight)
    lat_pad = jnp.zeros((8, CH), f32).at[:N_FACT].set(latent_emb)

    cor_buf, disen_w = _prep_call(d_pad, w_pad)
    cor = cor_buf[0, 0]

    ent = entity_emb
    usr = user_emb
    ent_res = entity_emb
    usr_res = user_emb

    for _ in range(N_HOPS):
        ent2 = ent.reshape(2 * N_ENT, CH // 2)
        kg_sums = _kg_call(ent2, kg_pack, w2)
        ui_sums = _ui_call(ent2, ui_pack)
        ent_new, ent_res = _ent_epi(
            kg_sums[0, :N_ENT], kg_sums[1, :N_ENT], ent_res)
        usr_new, usr_res = _usr_epi(
            ui_sums[0, :N_USERS], ui_sums[1, :N_USERS],
            usr, lat_pad, disen_w, usr_res)
        ent = ent_new
        usr = usr_new

    return ent_res, usr_res, cor


# speedup vs baseline: 1.8322x; 1.8322x over previous
import functools

import jax
import jax.numpy as jnp
from jax import lax
from jax.experimental import pallas as pl
from jax.experimental.pallas import tpu as pltpu
from jax.experimental.pallas import tpu_sc as plsc

N_ENT = 10000
N_USERS = 20000
CH = 128
N_EDGES = 320000
NNZ = 500000
N_REL = 10
N_FACT = 4
N_HOPS = 2

NC = 2
NS = 16
L = 16
NW = NC * NS

KG_CPS = 160
E_PAD = NS * KG_CPS * 128
KGW = 72
ACC_E = 10240
UI_CPS = 248
NNZ_PAD = NS * UI_CPS * 128
ACC_U = 20480
WREP = 128
NB = 2

def _sc_mesh():
    return plsc.VectorSubcoreMesh(core_axis_name="c", subcore_axis_name="s",
                                  num_cores=NC, num_subcores=NS)


def _kg_body(ent9_h, pack_h, out_h,
             pack, rows, stage, hbuf, acc, *sems):
    c = lax.axis_index("c")
    s = lax.axis_index("s")
    si, sg, ss = sems[0:NB], sems[NB:2 * NB], sems[2 * NB:3 * NB]
    z16 = jnp.zeros((L,), jnp.float32)
    iota16 = lax.iota(jnp.int32, L)

    @plsc.parallel_loop(0, 128, unroll=4)
    def _z1(r):
        for u in range(4):
            stage[0, r, pl.ds(u * L, L)] = z16
        stage[0, r, pl.ds(56, L)] = z16

    r0 = s * (ACC_E // NS)

    def z3(k, _):
        pltpu.sync_copy(stage.at[0], acc.at[pl.ds(r0 + k * 128, 128)])
        return 0
    lax.fori_loop(0, ACC_E // NS // 128, z3, 0)

    onehot = jnp.where(iota16 == 8, 1.0, 0.0).astype(jnp.float32)

    @plsc.parallel_loop(0, 128, unroll=8)
    def _z4(r):
        for b in range(NB):
            stage[b, r, pl.ds(56, L)] = onehot

    base = s * KG_CPS

    def issue_i(ci, b):
        pltpu.async_copy(pack_h.at[ci], pack.at[b], si[b])

    def wait_i(b):
        pltpu.make_async_copy(pack_h.at[0], pack.at[b], si[b]).wait()

    def fix_and_g(b, h):
        for g in range(8):
            sl = pl.ds(g * L, L)
            pack[b, 0, sl] = pack[b, 0, sl] + c
            hbuf[h, sl] = pack[b, 1, sl]
        pltpu.async_copy(ent9_h.at[pack.at[b].at[0]], rows.at[b], sg[b])

    def wait_g(b):
        pltpu.make_async_copy(ent9_h.at[pack.at[b].at[0]], rows.at[b],
                              sg[b]).wait()

    def compute_s(b, h):
        @plsc.parallel_loop(0, 128, unroll=4)
        def _m(r):
            for u in range(4):
                sl = pl.ds(u * L, L)
                stage[b, r, sl] = rows[b, r, sl]
        pltpu.async_copy(stage.at[b], acc.at[hbuf.at[h]], ss[b], add=True)

    def wait_s(b, h):
        pltpu.make_async_copy(stage.at[b], acc.at[hbuf.at[h]], ss[b]).wait()

    issue_i(base, 0)
    plsc.subcore_barrier()
    wait_i(0)
    fix_and_g(0, 0)
    issue_i(base + 1, 1)

    def outer(jj, _):
        for b4 in range(4):
            j = jj * 4 + b4
            b = b4 % NB
            b1 = (b + 1) % NB
            h1 = (b4 + 1) % 4

            @pl.when(j + 1 < KG_CPS)
            def _():
                wait_i(b1)
                fix_and_g(b1, h1)
            wait_g(b)

            @pl.when(j + NB < KG_CPS)
            def _():
                issue_i(base + j + NB, b)

            @pl.when(j >= NB)
            def _():
                wait_s(b, b4)
            compute_s(b, b4)
        return 0
    lax.fori_loop(0, KG_CPS // 4, outer, 0)
    wait_s(0, 0)
    wait_s(1, 1)
    plsc.subcore_barrier()
    nrow = ACC_E // NS
    pltpu.sync_copy(acc.at[pl.ds(r0, nrow)], out_h.at[c, pl.ds(r0, nrow)])


@functools.lru_cache(maxsize=None)
def _get_kg_call():
  return pl.kernel(
    _kg_body,
    out_type=jax.ShapeDtypeStruct((NC, ACC_E, KGW), jnp.float32),
    mesh=_sc_mesh(),
    compiler_params=pltpu.CompilerParams(use_tc_tiling_on_sc=False,
                                         needs_layout_passes=False),
    scratch_types=[
        pltpu.VMEM((NB, 2, 128), jnp.int32),
        pltpu.VMEM((NB, 128, CH // 2), jnp.float32),
        pltpu.VMEM((NB, 128, KGW), jnp.float32),
        pltpu.VMEM((4, 128), jnp.int32),
        pltpu.VMEM_SHARED((ACC_E, KGW), jnp.float32),
    ] + [pltpu.SemaphoreType.DMA] * (3 * NB),
  )


def _kg_call(*args):
    return _get_kg_call()(*args)


def _ui_body(ent2_h, pack_h, out_h,
             pack, grows, stage, hbuf, vbuf, acc, *sems):
    c = lax.axis_index("c")
    s = lax.axis_index("s")
    si, sg, ss = sems[0:NB], sems[NB:2 * NB], sems[2 * NB:3 * NB]
    z16 = jnp.zeros((L,), jnp.float32)

    @plsc.parallel_loop(0, 128, unroll=4)
    def _z1(r):
        for u in range(4):
            stage[0, r, pl.ds(u * L, L)] = z16

    r0 = s * (ACC_U // NS)

    def z3(k, _):
        pltpu.sync_copy(stage.at[0], acc.at[pl.ds(r0 + k * 128, 128)])
        return 0
    lax.fori_loop(0, ACC_U // NS // 128, z3, 0)

    base = s * UI_CPS

    def issue_i(ci, b):
        pltpu.async_copy(pack_h.at[ci], pack.at[b], si[b])

    def wait_i(b):
        pltpu.make_async_copy(pack_h.at[0], pack.at[b], si[b]).wait()

    def fix_and_g(b, h):
        for g in range(8):
            sl = pl.ds(g * L, L)
            pack[b, 0, sl] = pack[b, 0, sl] + c
            hbuf[h, sl] = pack[b, 1, sl]
            vbuf[b, sl] = pack[b, 2, sl]
        pltpu.async_copy(ent2_h.at[pack.at[b].at[0]], grows.at[b], sg[b])

    def wait_g(b):
        pltpu.make_async_copy(ent2_h.at[pack.at[b].at[0]], grows.at[b],
                              sg[b]).wait()

    def compute_s(b, h):
        @plsc.parallel_loop(0, 128, unroll=4)
        def _m(e):
            bv = jnp.full((L,), b, jnp.int32)
            ev = jnp.full((L,), e, jnp.int32)
            v16 = plsc.bitcast(plsc.load_gather(vbuf, [bv, ev]), jnp.float32)
            for u in range(4):
                sl = pl.ds(u * L, L)
                stage[b, e, sl] = grows[b, e, sl] * v16
        pltpu.async_copy(stage.at[b], acc.at[hbuf.at[h]], ss[b], add=True)

    def wait_s(b, h):
        pltpu.make_async_copy(stage.at[b], acc.at[hbuf.at[h]], ss[b]).wait()

    issue_i(base, 0)
    wait_i(0)
    fix_and_g(0, 0)
    issue_i(base + 1, 1)
    plsc.subcore_barrier()

    def outer(jj, _):
        for b4 in range(4):
            j = jj * 4 + b4
            b = b4 % NB
            b1 = (b + 1) % NB
            h1 = (b4 + 1) % 4

            @pl.when(j + 1 < UI_CPS)
            def _():
                wait_i(b1)
                fix_and_g(b1, h1)
            wait_g(b)

            @pl.when(j + NB < UI_CPS)
            def _():
                issue_i(base + j + NB, b)

            @pl.when(j >= NB)
            def _():
                wait_s(b, b4)
            compute_s(b, b4)
        return 0
    lax.fori_loop(0, UI_CPS // 4, outer, 0)
    wait_s(0, 0)
    wait_s(1, 1)
    plsc.subcore_barrier()
    nrow = ACC_U // NS
    pltpu.sync_copy(acc.at[pl.ds(r0, nrow)], out_h.at[c, pl.ds(r0, nrow)])


@functools.lru_cache(maxsize=None)
def _get_ui_call():
  return pl.kernel(
    _ui_body,
    out_type=jax.ShapeDtypeStruct((NC, ACC_U, CH // 2), jnp.float32),
    mesh=_sc_mesh(),
    compiler_params=pltpu.CompilerParams(use_tc_tiling_on_sc=False,
                                         needs_layout_passes=False),
    scratch_types=[
        pltpu.VMEM((NB, 3, 128), jnp.int32),
        pltpu.VMEM((NB, 128, CH // 2), jnp.float32),
        pltpu.VMEM((NB, 128, CH // 2), jnp.float32),
        pltpu.VMEM((4, 128), jnp.int32),
        pltpu.VMEM((NB, 128), jnp.int32),
        pltpu.VMEM_SHARED((ACC_U, CH // 2), jnp.float32),
    ] + [pltpu.SemaphoreType.DMA] * (3 * NB),
  )


def _ui_call(*args):
    return _get_ui_call()(*args)


def _prep_body(d_ref, w_ref, cor_ref, dw_ref):
    d = d_ref[...]
    nrm = jnp.sqrt(jnp.sum(d * d, axis=1, keepdims=True))
    dn = d / jnp.maximum(nrm, 1e-12)
    sim = jnp.dot(dn, dn.T, preferred_element_type=jnp.float32)
    r8 = lax.broadcasted_iota(jnp.int32, (8, 8), 0)
    c8 = lax.broadcasted_iota(jnp.int32, (8, 8), 1)
    cor = jnp.sum(jnp.where(c8 > r8, sim, 0.0))
    rr = lax.broadcasted_iota(jnp.int32, (8, CH), 0)
    cc = lax.broadcasted_iota(jnp.int32, (8, CH), 1)
    cor_ref[...] = jnp.where((rr == 0) & (cc == 0), cor, 0.0)
    logits = jnp.where(cc < (N_REL - 1), d, -jnp.inf)
    m = jnp.max(logits, axis=1, keepdims=True)
    e = jnp.exp(logits - m)
    sm = e / jnp.sum(e, axis=1, keepdims=True)
    dw_ref[...] = jnp.dot(sm[:, :16], w_ref[...],
                          preferred_element_type=jnp.float32)


def _prep_call(d_pad, w_pad):
    return pl.pallas_call(
        _prep_body,
        out_shape=[jax.ShapeDtypeStruct((8, CH), jnp.float32),
                   jax.ShapeDtypeStruct((8, CH), jnp.float32)],
    )(d_pad, w_pad)


_EBLK = 1000
_UBLK = 1000


def _ent_epi_body(s0_ref, s1_ref, res_ref, ent_ref, out_ref):
    h0 = s0_ref[...]
    h1 = s1_ref[...]
    sums = jnp.concatenate([h0[:, :CH // 2], h1[:, :CH // 2]], axis=1)
    cnt = h0[:, CH // 2:CH // 2 + 1]
    agg = sums / jnp.maximum(cnt, 1.0)
    nrm = jnp.sqrt(jnp.sum(agg * agg, axis=1, keepdims=True))
    e = agg / jnp.maximum(nrm, 1e-12)
    ent_ref[...] = e
    out_ref[...] = res_ref[...] + e


def _ent_epi(s0, s1, res):
    g = N_ENT // _EBLK
    bs = lambda w: pl.BlockSpec((_EBLK, w), lambda i: (i, 0))
    return pl.pallas_call(
        _ent_epi_body,
        grid=(g,),
        in_specs=[bs(KGW), bs(KGW), bs(CH)],
        out_specs=[bs(CH), bs(CH)],
        out_shape=[jax.ShapeDtypeStruct((N_ENT, CH), jnp.float32),
                   jax.ShapeDtypeStruct((N_ENT, CH), jnp.float32)],
    )(s0, s1, res)


def _usr_epi_body(ua0_ref, ua1_ref, usr_ref, lat_ref, dw_ref, res_ref,
                  unew_ref, out_ref):
    ua = jnp.concatenate([ua0_ref[...], ua1_ref[...]], axis=1)
    logits = jnp.dot(usr_ref[...], lat_ref[...].T,
                     preferred_element_type=jnp.float32)
    c8 = lax.broadcasted_iota(jnp.int32, (_UBLK, 8), 1)
    lg = jnp.where(c8 < N_FACT, logits, -jnp.inf)
    m = jnp.max(lg, axis=1, keepdims=True)
    e = jnp.exp(lg - m)
    score = e / jnp.sum(e, axis=1, keepdims=True)
    factor = jnp.dot(score, dw_ref[...], preferred_element_type=jnp.float32)
    agg = ua * (1.0 + factor)
    nrm = jnp.sqrt(jnp.sum(agg * agg, axis=1, keepdims=True))
    u = agg / jnp.maximum(nrm, 1e-12)
    unew_ref[...] = u
    out_ref[...] = res_ref[...] + u


def _usr_epi(ua0, ua1, usr, lat_pad, dw, res):
    g = N_USERS // _UBLK
    bs = lambda w: pl.BlockSpec((_UBLK, w), lambda i: (i, 0))
    fs = pl.BlockSpec((8, CH), lambda i: (0, 0))
    return pl.pallas_call(
        _usr_epi_body,
        grid=(g,),
        in_specs=[bs(CH // 2), bs(CH // 2), bs(CH), fs, fs, bs(CH)],
        out_specs=[bs(CH), bs(CH)],
        out_shape=[jax.ShapeDtypeStruct((N_USERS, CH), jnp.float32),
                   jax.ShapeDtypeStruct((N_USERS, CH), jnp.float32)],
    )(ua0, ua1, usr, lat_pad, dw, res)


def kernel(user_emb, entity_emb, latent_emb, edge_index, edge_type,
           inter_edge, inter_edge_w, mat_rows, mat_cols, mat_vals,
           weight, disen_weight_att):
    f32 = jnp.float32
    i32 = jnp.int32

    epad = E_PAD - N_EDGES
    head_p = jnp.concatenate(
        [edge_index[0], N_ENT + (jnp.arange(epad, dtype=i32) % L)])
    tail_p = jnp.concatenate([edge_index[1], jnp.zeros((epad,), i32)])
    ety_p = jnp.concatenate([edge_type, jnp.ones((epad,), i32)])
    gt_p = (ety_p - 1) * (2 * N_ENT) + 2 * tail_p
    kg_pack = (jnp.stack([gt_p, head_p], axis=0)
               .reshape(2, E_PAD // 128, 128).transpose(1, 0, 2))

    npad = NNZ_PAD - NNZ
    cols2_p = 2 * jnp.concatenate([mat_cols, jnp.zeros((npad,), i32)])
    urow_p = jnp.concatenate(
        [mat_rows, N_USERS + (jnp.arange(npad, dtype=i32) % L)])
    vals_p = jnp.concatenate([mat_vals, jnp.zeros((npad,), f32)])
    ui_pack = (jnp.stack([cols2_p, urow_p,
                          lax.bitcast_convert_type(vals_p, i32)], axis=0)
               .reshape(3, NNZ_PAD // 128, 128).transpose(1, 0, 2))

    d_pad = jnp.zeros((8, CH), f32).at[:N_FACT, :N_REL - 1].set(disen_weight_att)
    w_pad = jnp.zeros((16, CH), f32).at[:N_REL - 1].set(weight)
    lat_pad = jnp.zeros((8, CH), f32).at[:N_FACT].set(latent_emb)

    cor_buf, disen_w = _prep_call(d_pad, w_pad)
    cor = cor_buf[0, 0]

    ent = entity_emb
    usr = user_emb
    ent_res = entity_emb
    usr_res = user_emb

    for _ in range(N_HOPS):
        ent2 = ent.reshape(2 * N_ENT, CH // 2)
        ent9 = (weight[:, None, :] * ent[None, :, :]).reshape(
            (N_REL - 1) * 2 * N_ENT, CH // 2)
        kg_sums = _kg_call(ent9, kg_pack)
        ui_sums = _ui_call(ent2, ui_pack)
        ent_new, ent_res = _ent_epi(
            kg_sums[0, :N_ENT], kg_sums[1, :N_ENT], ent_res)
        usr_new, usr_res = _usr_epi(
            ui_sums[0, :N_USERS], ui_sums[1, :N_USERS],
            usr, lat_pad, disen_w, usr_res)
        ent = ent_new
        usr = usr_new

    return ent_res, usr_res, cor
